# baseline (speedup 1.0000x reference)
import functools

import jax
import jax.numpy as jnp
from jax import lax
from jax.experimental import pallas as pl
from jax.experimental.pallas import tpu as pltpu
from jax.experimental.pallas import tpu_sc as plsc

B = 16384
DIM = 64
NC = 2
NS = 16
L = 16
NW = NC * NS
BPW = B // NW
CHUNK = 32
NCHUNK = BPW // CHUNK
PASS = 256
NPASS = BPW // PASS
CPP = PASS // CHUNK
GPP = PASS // L


def _body(trip_hbm, nodes_hbm, rel_hbm, out_hbm,
          trip_v, idx_s, idx_p, idx_o,
          rows_s0, rows_p0, rows_o0, rows_s1, rows_p1, rows_o1,
          scores_v, sem):
    wid = lax.axis_index("s") * NC + lax.axis_index("c")
    base = wid * BPW

    pltpu.sync_copy(trip_hbm.at[pl.ds(base * 3, BPW * 3)], trip_v)

    iota = lax.broadcasted_iota(jnp.int32, (L,), 0)

    for m in range(BPW // L):
        flat = (m * L + iota) * 3
        r = m // (CHUNK // L)
        off = (m % (CHUNK // L)) * L
        idx_s[r, pl.ds(off, L)] = plsc.load_gather(trip_v, [flat])
        idx_p[r, pl.ds(off, L)] = plsc.load_gather(trip_v, [flat + 1])
        idx_o[r, pl.ds(off, L)] = plsc.load_gather(trip_v, [flat + 2])

    rows = [(rows_s0, rows_p0, rows_o0), (rows_s1, rows_p1, rows_o1)]

    copies = [[] for _ in range(NPASS)]
    for t in range(NPASS):
        rs, rp, ro = rows[t]
        for k in range(CPP):
            dst = pl.ds(k * CHUNK, CHUNK)
            kk = t * CPP + k
            copies[t].append(pltpu.async_copy(nodes_hbm.at[idx_s.at[kk]],
                                              rs.at[dst], sem))
            copies[t].append(pltpu.async_copy(rel_hbm.at[idx_p.at[kk]],
                                              rp.at[dst], sem))
            copies[t].append(pltpu.async_copy(nodes_hbm.at[idx_o.at[kk]],
                                              ro.at[dst], sem))

    for t in range(NPASS):
        rs, rp, ro = rows[t]
        for c in copies[t]:
            c.wait()

        def group(g, carry):
            lrow = g * L + iota
            acc = jnp.zeros((L,), jnp.float32)
            for j in range(DIM):
                col = jnp.full((L,), j, jnp.int32)
                sc = plsc.load_gather(rs, [lrow, col])
                pc = plsc.load_gather(rp, [lrow, col])
                oc = plsc.load_gather(ro, [lrow, col])
                acc = acc + sc * pc * oc
            scores_v[pl.ds(t * PASS + g * L, L)] = acc
            return carry

        lax.fori_loop(0, GPP, group, None)

    pltpu.sync_copy(scores_v, out_hbm.at[pl.ds(base, BPW)])


@functools.partial(
    pl.kernel,
    out_type=jax.ShapeDtypeStruct((B,), jnp.float32),
    mesh=plsc.VectorSubcoreMesh(core_axis_name="c", subcore_axis_name="s",
                                num_cores=NC, num_subcores=NS),
    scratch_types=[
        pltpu.VMEM((BPW * 3,), jnp.int32),
        pltpu.VMEM((NCHUNK, CHUNK), jnp.int32),
        pltpu.VMEM((NCHUNK, CHUNK), jnp.int32),
        pltpu.VMEM((NCHUNK, CHUNK), jnp.int32),
        pltpu.VMEM((PASS, DIM), jnp.float32),
        pltpu.VMEM((PASS, DIM), jnp.float32),
        pltpu.VMEM((PASS, DIM), jnp.float32),
        pltpu.VMEM((PASS, DIM), jnp.float32),
        pltpu.VMEM((PASS, DIM), jnp.float32),
        pltpu.VMEM((PASS, DIM), jnp.float32),
        pltpu.VMEM((BPW,), jnp.float32),
        pltpu.SemaphoreType.DMA,
    ],
    compiler_params=pltpu.CompilerParams(needs_layout_passes=False,
                                         use_tc_tiling_on_sc=False),
)
def _distmult_sc(trip_hbm, nodes_hbm, rel_hbm, out_hbm, *scratch):
    _body(trip_hbm, nodes_hbm, rel_hbm, out_hbm, *scratch)


def kernel(triples, nodes, relations):
    return _distmult_sc(triples.reshape(-1), nodes, relations)


# speedup vs baseline: 1.0020x; 1.0020x over previous
import functools

import jax
import jax.numpy as jnp
from jax import lax
from jax.experimental import pallas as pl
from jax.experimental.pallas import tpu as pltpu
from jax.experimental.pallas import tpu_sc as plsc

B = 16384
DIM = 64
NC = 2
NS = 16
L = 16
NW = NC * NS
BPW = B // NW
CHUNK = 32
NCHUNK = BPW // CHUNK
PASS = 256
NPASS = BPW // PASS
CPP = PASS // CHUNK
GPP = PASS // L


def _body(trip_hbm, nodes_hbm, rel_hbm, out_hbm,
          trip_v, idx_s, idx_p, idx_o,
          rows_s0, rows_p0, rows_o0, rows_s1, rows_p1, rows_o1,
          scores_v, sem):
    wid = lax.axis_index("s") * NC + lax.axis_index("c")
    base = wid * BPW

    pltpu.sync_copy(trip_hbm.at[pl.ds(base * 3, BPW * 3)], trip_v)

    iota = lax.broadcasted_iota(jnp.int32, (L,), 0)

    for m in range(BPW // L):
        flat = (m * L + iota) * 3
        r = m // (CHUNK // L)
        off = (m % (CHUNK // L)) * L
        idx_s[r, pl.ds(off, L)] = plsc.load_gather(trip_v, [flat])
        idx_p[r, pl.ds(off, L)] = plsc.load_gather(trip_v, [flat + 1])
        idx_o[r, pl.ds(off, L)] = plsc.load_gather(trip_v, [flat + 2])

    rows = [(rows_s0, rows_p0, rows_o0), (rows_s1, rows_p1, rows_o1)]

    copies = [[] for _ in range(NPASS)]
    for t in range(NPASS):
        rs, rp, ro = rows[t]
        for k in range(CPP):
            kk = t * CPP + k
            for h in range(CHUNK // L):
                dst = pl.ds(k * CHUNK + h * L, L)
                ssl = pl.ds(h * L, L)
                vs = idx_s[kk, ssl]
                vp = idx_p[kk, ssl]
                vo = idx_o[kk, ssl]
                copies[t].append(pltpu.async_copy(nodes_hbm.at[vs],
                                                  rs.at[dst], sem))
                copies[t].append(pltpu.async_copy(rel_hbm.at[vp],
                                                  rp.at[dst], sem))
                copies[t].append(pltpu.async_copy(nodes_hbm.at[vo],
                                                  ro.at[dst], sem))

    for t in range(NPASS):
        rs, rp, ro = rows[t]
        for c in copies[t]:
            c.wait()

        def group(g, carry):
            lrow = g * L + iota
            acc = jnp.zeros((L,), jnp.float32)
            for j in range(DIM):
                col = jnp.full((L,), j, jnp.int32)
                sc = plsc.load_gather(rs, [lrow, col])
                pc = plsc.load_gather(rp, [lrow, col])
                oc = plsc.load_gather(ro, [lrow, col])
                acc = acc + sc * pc * oc
            scores_v[pl.ds(t * PASS + g * L, L)] = acc
            return carry

        lax.fori_loop(0, GPP, group, None)

    pltpu.sync_copy(scores_v, out_hbm.at[pl.ds(base, BPW)])


@functools.partial(
    pl.kernel,
    out_type=jax.ShapeDtypeStruct((B,), jnp.float32),
    mesh=plsc.VectorSubcoreMesh(core_axis_name="c", subcore_axis_name="s",
                                num_cores=NC, num_subcores=NS),
    scratch_types=[
        pltpu.VMEM((BPW * 3,), jnp.int32),
        pltpu.VMEM((NCHUNK, CHUNK), jnp.int32),
        pltpu.VMEM((NCHUNK, CHUNK), jnp.int32),
        pltpu.VMEM((NCHUNK, CHUNK), jnp.int32),
        pltpu.VMEM((PASS, DIM), jnp.float32),
        pltpu.VMEM((PASS, DIM), jnp.float32),
        pltpu.VMEM((PASS, DIM), jnp.float32),
        pltpu.VMEM((PASS, DIM), jnp.float32),
        pltpu.VMEM((PASS, DIM), jnp.float32),
        pltpu.VMEM((PASS, DIM), jnp.float32),
        pltpu.VMEM((BPW,), jnp.float32),
        pltpu.SemaphoreType.DMA,
    ],
    compiler_params=pltpu.CompilerParams(needs_layout_passes=False,
                                         use_tc_tiling_on_sc=False),
)
def _distmult_sc(trip_hbm, nodes_hbm, rel_hbm, out_hbm, *scratch):
    _body(trip_hbm, nodes_hbm, rel_hbm, out_hbm, *scratch)


def kernel(triples, nodes, relations):
    return _distmult_sc(triples.reshape(-1), nodes, relations)
